# Initial kernel scaffold; baseline (speedup 1.0000x reference)
import jax
import jax.numpy as jnp
from jax.experimental import pallas as pl


def kernel(x, word_table, freq_table, phase_table):
    raise NotImplementedError("write your pallas kernel here")


# speedup vs baseline: 13.9513x; 13.9513x over previous
import functools

import jax
import jax.numpy as jnp
from jax import lax
from jax.experimental import pallas as pl
from jax.experimental.pallas import tpu as pltpu
from jax.experimental.pallas import tpu_sc as plsc

D_HALF = 64
D_MODEL = 128
LANES = 16
CHUNK = 128

TWO_PI = 6.283185307179586
INV_TWO_PI = 0.15915494309189535
MAGIC = 12582912.0

_COS_C = (0.998987151976084, -0.4962486273058178,
          0.039522302756833556, -0.0009928615940640903)
_SIN_C = (0.9998824651862409, -0.1662326327675863,
          0.00808644586820862, -0.00015325191256653205)


def _sincos_chain(pv, f, bb, a):
    ph = pv * f + bb
    t = (ph * jnp.float32(INV_TWO_PI) + jnp.float32(MAGIC)) - jnp.float32(MAGIC)
    r = ph - t * jnp.float32(TWO_PI)
    x2 = r * r
    c = jnp.float32(_COS_C[3])
    for k in (2, 1, 0):
        c = c * x2 + jnp.float32(_COS_C[k])
    s = jnp.float32(_SIN_C[3])
    for k in (2, 1, 0):
        s = s * x2 + jnp.float32(_SIN_C[k])
    return a * c, a * (s * r)


def _build(n_tokens, seq_len):
    info = plsc.get_sparse_core_info()
    nc, ns = info.num_cores, info.num_subcores
    nw = nc * ns
    assert n_tokens % (nw * CHUNK) == 0
    per_w = n_tokens // nw
    n_chunks = per_w // CHUNK
    assert n_chunks % 2 == 0

    mesh = plsc.VectorSubcoreMesh(core_axis_name="c", subcore_axis_name="s")
    vm = pltpu.VMEM

    @functools.partial(
        pl.kernel,
        mesh=mesh,
        out_type=jax.ShapeDtypeStruct((n_tokens, D_MODEL), jnp.float32),
        scratch_types=[
            vm((per_w,), jnp.int32),
            vm((2, CHUNK, D_HALF), jnp.float32),
            vm((2, CHUNK, D_HALF), jnp.float32),
            vm((2, CHUNK, D_HALF), jnp.float32),
            vm((2, CHUNK, D_MODEL), jnp.float32),
            pltpu.SemaphoreType.DMA,
            pltpu.SemaphoreType.DMA,
            pltpu.SemaphoreType.DMA,
            pltpu.SemaphoreType.DMA,
        ],
        compiler_params=pltpu.CompilerParams(use_tc_tiling_on_sc=False),
    )
    def kern(x_hbm, word_hbm, freq_hbm, phase_hbm, out_hbm,
             idx_v, amp_v, frq_v, bia_v, out_v, sem_g0, sem_g1, sem_o0, sem_o1):
        wid = lax.axis_index("s") * nc + lax.axis_index("c")
        base_w = wid * per_w
        sem_g = (sem_g0, sem_g1)
        sem_o = (sem_o0, sem_o1)

        pltpu.sync_copy(x_hbm.at[pl.ds(base_w, per_w)], idx_v)

        def idx_slice(ci):
            return idx_v.at[pl.ds(ci * CHUNK, CHUNK)]

        def start_gathers(ci, b):
            pltpu.async_copy(word_hbm.at[idx_slice(ci)], amp_v.at[b], sem_g[b])
            pltpu.async_copy(freq_hbm.at[idx_slice(ci)], frq_v.at[b], sem_g[b])
            pltpu.async_copy(phase_hbm.at[idx_slice(ci)], bia_v.at[b], sem_g[b])

        def wait_gathers(ci, b):
            pltpu.make_async_copy(word_hbm.at[idx_slice(ci)], amp_v.at[b], sem_g[b]).wait()
            pltpu.make_async_copy(freq_hbm.at[idx_slice(ci)], frq_v.at[b], sem_g[b]).wait()
            pltpu.make_async_copy(phase_hbm.at[idx_slice(ci)], bia_v.at[b], sem_g[b]).wait()

        def drain_out(ci, b):
            base = base_w + ci * CHUNK
            pltpu.make_async_copy(
                out_v.at[b], out_hbm.at[pl.ds(base, CHUNK)], sem_o[b]).wait()

        start_gathers(0, 0)

        def pair_body(cp, carry):
            for b in (0, 1):
                ci = cp * 2 + b

                @pl.when(ci + 1 < n_chunks)
                def _():
                    start_gathers(ci + 1, 1 - b)

                @pl.when(ci >= 2)
                def _():
                    drain_out(ci - 2, b)

                wait_gathers(ci, b)

                def tok_body(t2, tc):
                    chains = []
                    for u in (0, 1, 2, 3):
                        t = t2 * 4 + u
                        p = lax.rem(base_w + ci * CHUNK + t, seq_len) + 1
                        pv = jnp.full((LANES,), p.astype(jnp.float32))
                        for j in range(D_HALF // LANES):
                            sl = pl.ds(j * LANES, LANES)
                            chains.append((t, j, pv, frq_v[b, t, sl],
                                           bia_v[b, t, sl], amp_v[b, t, sl]))
                    results = [(t, j) + _sincos_chain(pv, f, bb, a)
                               for (t, j, pv, f, bb, a) in chains]
                    for t, j, oc, oi in results:
                        out_v[b, t, pl.ds(j * LANES, LANES)] = oc
                        out_v[b, t, pl.ds(D_HALF + j * LANES, LANES)] = oi
                    return tc

                lax.fori_loop(0, CHUNK // 4, tok_body, 0)
                base = base_w + ci * CHUNK
                pltpu.async_copy(out_v.at[b], out_hbm.at[pl.ds(base, CHUNK)], sem_o[b])
            return carry

        lax.fori_loop(0, n_chunks // 2, pair_body, 0)
        drain_out(n_chunks - 2, 0)
        drain_out(n_chunks - 1, 1)

    return kern


def kernel(x, word_table, freq_table, phase_table):
    b, length = x.shape
    n = b * length
    xf = x.reshape(n)
    out = _build(n, length)(xf, word_table, freq_table, phase_table)
    return out.reshape(b, length, D_MODEL)
